# baseline (speedup 1.0000x reference)
import functools

import jax
import jax.numpy as jnp
from jax import lax
from jax.experimental import pallas as pl
from jax.experimental.pallas import tpu as pltpu
from jax.experimental.pallas import tpu_sc as plsc

_N = 10000
_E = 320000
_D = 128
_NC = 2
_NS = 16
_NW = _NC * _NS
_EPW = _E // _NW
_K = 32
_NCH = _EPW // _K
_TAIL = _EPW - _NCH * _K
_NZC = -(-_N // _K)
_RCPT = -(-_NZC // _NS)
_NB = 4


def _sc_conv_body(node_hbm, packed_hbm, edge_hbm, out_hbm, *refs):
    packed_all = refs[0]
    srcv = refs[1:1 + _NB]
    dstv = refs[1 + _NB:1 + 2 * _NB]
    srcv_t = refs[1 + 2 * _NB]
    dstv_t = refs[2 + 2 * _NB]
    rows = refs[3 + 2 * _NB:3 + 3 * _NB]
    msg = refs[3 + 3 * _NB:3 + 4 * _NB]
    acc_sh = refs[3 + 4 * _NB]
    sems = refs[4 + 4 * _NB:]
    sem_g = sems[0:_NB]
    sem_e = sems[_NB:2 * _NB]
    sem_s = sems[2 * _NB:3 * _NB]

    c = lax.axis_index("c")
    s = lax.axis_index("s")
    w = s * _NC + c
    ebase = w * _EPW

    pltpu.sync_copy(packed_hbm.at[pl.ds(ebase, _EPW)], packed_all)

    def unpack(j, b):
        for g in range(_K // 16):
            v = packed_all[pl.ds(j * _K + g * 16, 16)]
            srcv[b][pl.ds(g * 16, 16)] = v & 0x3FFF
            dstv[b][pl.ds(g * 16, 16)] = lax.shift_right_logical(v, 14)

    def issue_loads(j, b):
        base = ebase + j * _K
        pltpu.async_copy(node_hbm.at[srcv[b]], rows[b], sem_g[b])
        pltpu.async_copy(edge_hbm.at[pl.ds(base, _K)], msg[b], sem_e[b])

    def wait_loads(j, b):
        base = ebase + j * _K
        pltpu.make_async_copy(node_hbm.at[srcv[b]], rows[b],
                              sem_g[b]).wait()
        pltpu.make_async_copy(edge_hbm.at[pl.ds(base, _K)], msg[b],
                              sem_e[b]).wait()

    def compute(b):
        m, x = msg[b], rows[b]

        def row2(r2, rc):
            for dr in range(2):
                r = r2 * 2 + dr
                for cc in range(_D // 16):
                    sl = pl.ds(cc * 16, 16)
                    m[r, sl] = jnp.maximum(m[r, sl] + x[r, sl], 0.0)
            return rc

        lax.fori_loop(0, _K // 2, row2, 0)

    def issue_scatter(b):
        pltpu.async_copy(msg[b], acc_sh.at[dstv[b]], sem_s[b], add=True)

    def wait_scatter(b):
        pltpu.make_async_copy(msg[b], acc_sh.at[dstv[b]], sem_s[b]).wait()

    unpack(0, 0)
    issue_loads(0, 0)
    unpack(1, 1)
    issue_loads(1, 1)

    def zrow(r, carry):
        for cc in range(_D // 16):
            rows[2][r, pl.ds(cc * 16, 16)] = jnp.zeros((16,), jnp.float32)
        return carry

    lax.fori_loop(0, _K, zrow, 0)

    def zchunk(i, carry):
        ch = s + i * _NS

        @pl.when(ch < _NZC)
        def _():
            off = jnp.minimum(ch * _K, _N - _K)
            pltpu.sync_copy(rows[2], acc_sh.at[pl.ds(off, _K)])

        return carry

    lax.fori_loop(0, _RCPT, zchunk, 0)
    plsc.subcore_barrier()

    def oct_(t, carry):
        for sstep in range(8):
            j = 8 * t + sstep
            b = sstep % _NB
            b2 = (sstep + 2) % _NB

            @pl.when(j >= 2)
            def _():
                wait_scatter(b2)

            @pl.when(j + 2 < _NCH)
            def _():
                unpack(j + 2, b2)
                issue_loads(j + 2, b2)

            wait_loads(j, b)
            compute(b)
            issue_scatter(b)
        return carry

    lax.fori_loop(0, _NCH // 8, oct_, 0)

    wait_scatter(2)
    v = packed_all[pl.ds(_NCH * _K, _TAIL)]
    srcv_t[...] = v & 0x3FFF
    dstv_t[...] = lax.shift_right_logical(v, 14)
    tbase = ebase + _NCH * _K
    pltpu.async_copy(node_hbm.at[srcv_t], rows[2].at[pl.ds(0, _TAIL)],
                     sem_g[2])
    pltpu.async_copy(edge_hbm.at[pl.ds(tbase, _TAIL)],
                     msg[2].at[pl.ds(0, _TAIL)], sem_e[2])
    wait_scatter(3)
    pltpu.make_async_copy(node_hbm.at[srcv_t], rows[2].at[pl.ds(0, _TAIL)],
                          sem_g[2]).wait()
    pltpu.make_async_copy(edge_hbm.at[pl.ds(tbase, _TAIL)],
                          msg[2].at[pl.ds(0, _TAIL)], sem_e[2]).wait()

    def rowt(r, rc):
        for cc in range(_D // 16):
            sl = pl.ds(cc * 16, 16)
            msg[2][r, sl] = jnp.maximum(msg[2][r, sl] + rows[2][r, sl], 0.0)
        return rc

    lax.fori_loop(0, _TAIL, rowt, 0)
    pltpu.async_copy(msg[2].at[pl.ds(0, _TAIL)], acc_sh.at[dstv_t],
                     sem_s[2], add=True)
    pltpu.make_async_copy(msg[2].at[pl.ds(0, _TAIL)], acc_sh.at[dstv_t],
                          sem_s[2]).wait()
    plsc.subcore_barrier()

    def ochunk(i, carry):
        ch = s + i * _NS

        @pl.when(ch < _NZC)
        def _():
            off = jnp.minimum(ch * _K, _N - _K)
            pltpu.sync_copy(acc_sh.at[pl.ds(off, _K)], msg[0])
            pltpu.sync_copy(msg[0], out_hbm.at[c, pl.ds(off, _K)])

        return carry

    lax.fori_loop(0, _RCPT, ochunk, 0)


@functools.cache
def _sc_conv():
    return functools.partial(
        pl.kernel,
        out_type=jax.ShapeDtypeStruct((_NC, _N, _D), jnp.float32),
        mesh=plsc.VectorSubcoreMesh(core_axis_name="c", subcore_axis_name="s",
                                    num_cores=_NC, num_subcores=_NS),
        scratch_types=(
            [pltpu.VMEM((_EPW,), jnp.int32)]
            + [pltpu.VMEM((_K,), jnp.int32) for _ in range(2 * _NB)]
            + [pltpu.VMEM((_TAIL,), jnp.int32) for _ in range(2)]
            + [pltpu.VMEM((_K, _D), jnp.float32) for _ in range(2 * _NB)]
            + [pltpu.VMEM_SHARED((_N, _D), jnp.float32)]
            + [pltpu.SemaphoreType.DMA for _ in range(3 * _NB)]
        ),
    )(_sc_conv_body)


def _tc_body(node_ref, agg_ref, w1_ref, b1_ref, w2_ref, b2_ref,
             gamma_ref, beta_ref, out_ref):
    h = node_ref[...] + agg_ref[0] + agg_ref[1]
    h = jnp.maximum(
        lax.dot_general(h, w1_ref[...], (((1,), (0,)), ((), ())),
                        preferred_element_type=jnp.float32) + b1_ref[...], 0.0)
    h = lax.dot_general(h, w2_ref[...], (((1,), (0,)), ((), ())),
                        preferred_element_type=jnp.float32) + b2_ref[...]
    mean = jnp.mean(h, axis=0, keepdims=True)
    var = jnp.mean(jnp.square(h - mean), axis=0, keepdims=True)
    out_ref[...] = ((h - mean) * lax.rsqrt(var + 1e-5) * gamma_ref[...]
                    + beta_ref[...])


_tc_finish = pl.pallas_call(
    _tc_body,
    out_shape=jax.ShapeDtypeStruct((_N, _D), jnp.float32),
)


def kernel(node_feats, edge_feats, W1, b1, W2, b2, gamma, beta, edge_index):
    src = edge_index[0]
    dst = edge_index[1]
    packed = jnp.bitwise_or(src, jnp.left_shift(dst, 14))
    agg2 = _sc_conv()(node_feats, packed, edge_feats)
    return _tc_finish(node_feats, agg2,
                      W1, b1.reshape(1, _D),
                      W2, b2.reshape(1, _D),
                      gamma.reshape(1, _D), beta.reshape(1, _D))
